# baseline (speedup 1.0000x reference)
import functools

import jax
import jax.numpy as jnp
from jax import lax
from jax.experimental import pallas as pl
from jax.experimental.pallas import tpu as pltpu
from jax.experimental.pallas import tpu_sc as plsc

NUM_EMBEDDINGS = 1000000
EMBEDDING_DIM = 64
TOTAL_VALUES = 106496

NC = 2
NS = 16
NW = NC * NS
BPW = TOTAL_VALUES // NW
R = 128
NCHUNKS = BPW // R
NPAIRS = NUM_EMBEDDINGS // 2

_mesh = plsc.VectorSubcoreMesh(core_axis_name="c", subcore_axis_name="s")


@functools.partial(
    pl.kernel,
    out_type=jax.ShapeDtypeStruct((TOTAL_VALUES, EMBEDDING_DIM), jnp.float32),
    mesh=_mesh,
    compiler_params=pltpu.CompilerParams(needs_layout_passes=False),
    scratch_types=[
        pltpu.VMEM((BPW,), jnp.int32),
        pltpu.VMEM((R,), jnp.int32),
        pltpu.VMEM((R, 2 * EMBEDDING_DIM), jnp.float32),
        pltpu.VMEM((R, EMBEDDING_DIM), jnp.float32),
        pltpu.SemaphoreType.DMA,
        pltpu.SemaphoreType.DMA,
    ],
)
def _sc_gather(table_hbm, idx_hbm, out_hbm, idx_v, pid_v, slab, stage,
               gsem, ssem):
    wid = lax.axis_index("s") * NC + lax.axis_index("c")
    base = wid * BPW
    pltpu.sync_copy(idx_hbm.at[pl.ds(base, BPW)], idx_v)

    lanes = lax.iota(jnp.int32, 16)

    @pl.loop(0, NCHUNKS)
    def _(c):
        cb = c * R
        for j in range(R // 16):
            v = idx_v[pl.ds(cb + j * 16, 16)]
            pid_v[pl.ds(j * 16, 16)] = lax.shift_right_logical(v, 1)
        pltpu.async_copy(table_hbm.at[pid_v], slab, gsem).wait()
        for j in range(R // 16):
            offs = (idx_v[pl.ds(cb + j * 16, 16)] & 1) * EMBEDDING_DIM
            for t in range(16):
                i = j * 16 + t
                d0 = jnp.full((16,), i, dtype=jnp.int32)
                d1 = offs[t] + lanes
                for k in range(EMBEDDING_DIM // 16):
                    vals = plsc.load_gather(slab, [d0, d1 + k * 16])
                    stage[i, pl.ds(k * 16, 16)] = vals
        off = pl.multiple_of(base + cb, R)
        pltpu.async_copy(stage, out_hbm.at[pl.ds(off, R)], ssem).wait()


def kernel(table, indices, offsets):
    del offsets
    t2 = table.reshape(NPAIRS, 2 * EMBEDDING_DIM)
    return _sc_gather(t2, indices.astype(jnp.int32))


# speedup vs baseline: 1.7006x; 1.7006x over previous
import functools

import jax
import jax.numpy as jnp
from jax import lax
from jax.experimental import pallas as pl
from jax.experimental.pallas import tpu as pltpu
from jax.experimental.pallas import tpu_sc as plsc

NUM_EMBEDDINGS = 1000000
EMBEDDING_DIM = 64
TOTAL_VALUES = 106496

NC = 2
NS = 16
NW = NC * NS
BPW = TOTAL_VALUES // NW
R = 32
NCHUNKS = BPW // R
NGROUPS = NUM_EMBEDDINGS // 8

_mesh = plsc.VectorSubcoreMesh(core_axis_name="c", subcore_axis_name="s")


@functools.partial(
    pl.kernel,
    out_type=jax.ShapeDtypeStruct((TOTAL_VALUES // 8, 8, EMBEDDING_DIM),
                                  jnp.float32),
    mesh=_mesh,
    compiler_params=pltpu.CompilerParams(needs_layout_passes=False),
    scratch_types=[
        pltpu.VMEM((BPW,), jnp.int32),
        pltpu.VMEM((R, 8, EMBEDDING_DIM), jnp.float32),
        pltpu.VMEM((R, 8, EMBEDDING_DIM), jnp.float32),
        pltpu.VMEM((R // 8, 8, EMBEDDING_DIM), jnp.float32),
        pltpu.VMEM((R // 8, 8, EMBEDDING_DIM), jnp.float32),
        pltpu.SemaphoreType.DMA,
        pltpu.SemaphoreType.DMA,
        pltpu.SemaphoreType.DMA,
        pltpu.SemaphoreType.DMA,
    ],
)
def _sc_gather(table_hbm, idx_hbm, out_hbm, idx_v,
               slab0, slab1, stage0, stage1, gsem0, gsem1, ssem0, ssem1):
    wid = lax.axis_index("s") * NC + lax.axis_index("c")
    base = wid * BPW
    gbase = wid * (BPW // 8)
    pltpu.sync_copy(idx_hbm.at[pl.ds(base, BPW)], idx_v)

    slabs = (slab0, slab1)
    stages = (stage0, stage1)
    gsems = (gsem0, gsem1)
    ssems = (ssem0, ssem1)
    lanes = lax.iota(jnp.int32, 16)

    def gather_start(q, h):
        for j in range(R // 16):
            gv = lax.shift_right_logical(
                idx_v[pl.ds(q * R + j * 16, 16)], 3)
            for t in range(16):
                i = j * 16 + t
                pltpu.async_copy(table_hbm.at[gv[t]], slabs[h].at[i],
                                 gsems[h])

    def gather_drain(h):
        pltpu.make_async_copy(table_hbm.at[pl.ds(0, R)], slabs[h],
                              gsems[h]).wait()

    def store_desc(q, h):
        off = pl.multiple_of(gbase + q * (R // 8), R // 8)
        return pltpu.make_async_copy(
            stages[h], out_hbm.at[pl.ds(off, R // 8)], ssems[h])

    gather_start(0, 0)
    gather_start(1, 1)

    @pl.loop(0, NCHUNKS, step=2)
    def _(c):
        for h in range(2):
            q = c + h
            gather_drain(h)
            @pl.when(q >= 2)
            def _():
                store_desc(q, h).wait()
            for j in range(R // 16):
                subs = idx_v[pl.ds(q * R + j * 16, 16)] & 7
                for t in range(16):
                    i = j * 16 + t
                    d0 = jnp.full((16,), i, dtype=jnp.int32)
                    d1 = jnp.full((16,), subs[t], dtype=jnp.int32)
                    for k in range(EMBEDDING_DIM // 16):
                        vals = plsc.load_gather(
                            slabs[h], [d0, d1, lanes + k * 16])
                        stages[h][i // 8, i % 8, pl.ds(k * 16, 16)] = vals
            store_desc(q, h).start()

            @pl.when(q + 2 < NCHUNKS)
            def _():
                gather_start(q + 2, h)

    store_desc(NCHUNKS - 2, 0).wait()
    store_desc(NCHUNKS - 1, 1).wait()


def kernel(table, indices, offsets):
    del offsets
    t3 = table.reshape(NGROUPS, 8, EMBEDDING_DIM)
    out = _sc_gather(t3, indices.astype(jnp.int32))
    return out.reshape(TOTAL_VALUES, EMBEDDING_DIM)


# speedup vs baseline: 1.7842x; 1.0492x over previous
import functools

import jax
import jax.numpy as jnp
from jax import lax
from jax.experimental import pallas as pl
from jax.experimental.pallas import tpu as pltpu
from jax.experimental.pallas import tpu_sc as plsc

NUM_EMBEDDINGS = 1000000
EMBEDDING_DIM = 64
TOTAL_VALUES = 106496

NC = 2
NS = 16
NW = NC * NS
BPW = TOTAL_VALUES // NW
R = 16
NCHUNKS = BPW // R
NBUF = 4
NGROUPS = NUM_EMBEDDINGS // 8

_mesh = plsc.VectorSubcoreMesh(core_axis_name="c", subcore_axis_name="s")


@functools.partial(
    pl.kernel,
    out_type=jax.ShapeDtypeStruct((TOTAL_VALUES // 8, 8, EMBEDDING_DIM),
                                  jnp.float32),
    mesh=_mesh,
    compiler_params=pltpu.CompilerParams(needs_layout_passes=False),
    scratch_types=(
        [pltpu.VMEM((BPW,), jnp.int32)]
        + [pltpu.VMEM((R, 8, EMBEDDING_DIM), jnp.float32)] * NBUF
        + [pltpu.VMEM((R // 8, 8, EMBEDDING_DIM), jnp.float32)] * 2
        + [pltpu.SemaphoreType.DMA] * (NBUF + 2)
    ),
)
def _sc_gather(table_hbm, idx_hbm, out_hbm, idx_v,
               slab0, slab1, slab2, slab3, stage0, stage1,
               gsem0, gsem1, gsem2, gsem3, ssem0, ssem1):
    wid = lax.axis_index("s") * NC + lax.axis_index("c")
    base = wid * BPW
    gbase = wid * (BPW // 8)
    pltpu.sync_copy(idx_hbm.at[pl.ds(base, BPW)], idx_v)

    slabs = (slab0, slab1, slab2, slab3)
    stages = (stage0, stage1)
    gsems = (gsem0, gsem1, gsem2, gsem3)
    ssems = (ssem0, ssem1)
    lanes = lax.iota(jnp.int32, 16)

    def gather_start(q, h):
        gv = lax.shift_right_logical(idx_v[pl.ds(q * R, 16)], 3)
        for t in range(16):
            pltpu.async_copy(table_hbm.at[gv[t]], slabs[h].at[t], gsems[h])

    def gather_drain(h):
        pltpu.make_async_copy(table_hbm.at[pl.ds(0, R)], slabs[h],
                              gsems[h]).wait()

    def store_desc(q, sh):
        off = pl.multiple_of(gbase + q * (R // 8), R // 8)
        return pltpu.make_async_copy(
            stages[sh], out_hbm.at[pl.ds(off, R // 8)], ssems[sh])

    for h in range(NBUF):
        gather_start(h, h)

    @pl.loop(0, NCHUNKS, step=NBUF)
    def _(c):
        for h in range(NBUF):
            q = c + h
            sh = h % 2
            gather_drain(h)
            @pl.when(q >= 2)
            def _():
                store_desc(q, sh).wait()
            subs = idx_v[pl.ds(q * R, 16)] & 7
            for t in range(16):
                d0 = jnp.full((16,), t, dtype=jnp.int32)
                d1 = jnp.full((16,), subs[t], dtype=jnp.int32)
                for k in range(EMBEDDING_DIM // 16):
                    vals = plsc.load_gather(
                        slabs[h], [d0, d1, lanes + k * 16])
                    stages[sh][t // 8, t % 8, pl.ds(k * 16, 16)] = vals
            store_desc(q, sh).start()

            @pl.when(q + NBUF < NCHUNKS)
            def _():
                gather_start(q + NBUF, h)

    store_desc(NCHUNKS - 2, 0).wait()
    store_desc(NCHUNKS - 1, 1).wait()


def kernel(table, indices, offsets):
    del offsets
    t3 = table.reshape(NGROUPS, 8, EMBEDDING_DIM)
    out = _sc_gather(t3, indices.astype(jnp.int32))
    return out.reshape(TOTAL_VALUES, EMBEDDING_DIM)


# speedup vs baseline: 1.7883x; 1.0023x over previous
import functools

import jax
import jax.numpy as jnp
from jax import lax
from jax.experimental import pallas as pl
from jax.experimental.pallas import tpu as pltpu
from jax.experimental.pallas import tpu_sc as plsc

NUM_EMBEDDINGS = 1000000
EMBEDDING_DIM = 64
TOTAL_VALUES = 106496

NC = 2
NS = 16
NW = NC * NS
BPW = TOTAL_VALUES // NW
R = 16
NCHUNKS = BPW // R
NBUF = 4
NGROUPS = NUM_EMBEDDINGS // 8

_mesh = plsc.VectorSubcoreMesh(core_axis_name="c", subcore_axis_name="s")


@functools.partial(
    pl.kernel,
    out_type=jax.ShapeDtypeStruct((TOTAL_VALUES // 8, 8, EMBEDDING_DIM),
                                  jnp.float32),
    mesh=_mesh,
    compiler_params=pltpu.CompilerParams(needs_layout_passes=False),
    scratch_types=(
        [pltpu.VMEM((BPW,), jnp.int32)]
        + [pltpu.VMEM((R, 8, EMBEDDING_DIM), jnp.float32)] * NBUF
        + [pltpu.VMEM((R // 8, 8, EMBEDDING_DIM), jnp.float32)] * 2
        + [pltpu.SemaphoreType.DMA] * (NBUF + 2)
    ),
)
def _sc_gather(table_hbm, idx_hbm, out_hbm, idx_v,
               slab0, slab1, slab2, slab3, stage0, stage1,
               gsem0, gsem1, gsem2, gsem3, ssem0, ssem1):
    wid = lax.axis_index("s") * NC + lax.axis_index("c")
    base = wid * BPW
    gbase = wid * (BPW // 8)
    pltpu.sync_copy(idx_hbm.at[pl.ds(base, BPW)], idx_v)

    slabs = (slab0, slab1, slab2, slab3)
    stages = (stage0, stage1)
    gsems = (gsem0, gsem1, gsem2, gsem3)
    ssems = (ssem0, ssem1)
    lanes = lax.iota(jnp.int32, 16)

    def gather_start(q, h):
        gv = lax.shift_right_logical(idx_v[pl.ds(q * R, 16)], 3)
        for t in range(16):
            pltpu.async_copy(table_hbm.at[gv[t]], slabs[h].at[t], gsems[h])

    def gather_drain(h):
        pltpu.make_async_copy(table_hbm.at[pl.ds(0, R)], slabs[h],
                              gsems[h]).wait()

    def store_desc(q, sh):
        off = pl.multiple_of(gbase + q * (R // 8), R // 8)
        return pltpu.make_async_copy(
            stages[sh], out_hbm.at[pl.ds(off, R // 8)], ssems[sh])

    for h in range(NBUF):
        gather_start(h, h)

    @pl.loop(0, NCHUNKS, step=NBUF)
    def _(c):
        for h in range(NBUF):
            q = c + h
            sh = h % 2
            gather_drain(h)
            @pl.when(q >= 2)
            def _():
                store_desc(q, sh).wait()
            subs = idx_v[pl.ds(q * R, 16)] & 7
            for t in range(16):
                for k in range(EMBEDDING_DIM // 16):
                    vals = slabs[h][t, subs[t], pl.ds(k * 16, 16)]
                    stages[sh][t // 8, t % 8, pl.ds(k * 16, 16)] = vals
            store_desc(q, sh).start()

            @pl.when(q + NBUF < NCHUNKS)
            def _():
                gather_start(q + NBUF, h)

    store_desc(NCHUNKS - 2, 0).wait()
    store_desc(NCHUNKS - 1, 1).wait()


def kernel(table, indices, offsets):
    del offsets
    t3 = table.reshape(NGROUPS, 8, EMBEDDING_DIM)
    out = _sc_gather(t3, indices.astype(jnp.int32))
    return out.reshape(TOTAL_VALUES, EMBEDDING_DIM)
